# baseline (speedup 1.0000x reference)
import functools

import jax
import jax.numpy as jnp
from jax import lax
from jax.experimental import pallas as pl
from jax.experimental.pallas import tpu as pltpu
from jax.experimental.pallas import tpu_sc as plsc

D = 64
SCALE = 8.0
HIST = 200
TAIL = HIST - 128


def _split_body(tok_ref, t0_ref, t1_ref):
    x = tok_ref[...]
    t0_ref[...] = x[:, :128]
    t1_ref[...] = x[:, HIST - 128:HIST]


def _gather_body(t0_hbm, t1_hbm, table_hbm, out_hbm, idx0_v, idx1_v, rows_v,
                 comp_v, sem, *, rows_per_w, nc):
    wid = lax.axis_index("s") * nc + lax.axis_index("c")
    b0 = pl.multiple_of(wid * rows_per_w, 8)
    pltpu.sync_copy(t0_hbm.at[pl.ds(b0, rows_per_w)], idx0_v)
    pltpu.sync_copy(t1_hbm.at[pl.ds(b0, rows_per_w)], idx1_v)

    def pair_body(jp, carry):
        copies = []
        for h in range(2):
            j = 2 * jp + h
            base = h * HIST
            copies.append(pltpu.async_copy(
                table_hbm.at[idx0_v.at[j]],
                rows_v.at[pl.ds(base, 128)], sem))
            copies.append(pltpu.async_copy(
                table_hbm.at[idx1_v.at[j, pl.ds(128 - TAIL, TAIL)]],
                rows_v.at[pl.ds(base + 128, TAIL)], sem))
        for c in copies:
            c.wait()

        def repack_body(r, carry2):
            for k in range(8):
                src = (2 * r + k // 4, pl.ds((k % 4) * 16, 16))
                comp_v[r, pl.ds(k * 16, 16)] = rows_v[src] * SCALE
            return carry2

        lax.fori_loop(0, HIST, repack_body, 0, unroll=2)

        off = pl.multiple_of(100 * b0 + HIST * jp, 8)
        pltpu.sync_copy(comp_v, out_hbm.at[pl.ds(off, HIST)])
        return carry

    lax.fori_loop(0, rows_per_w // 2, pair_body, 0)


def kernel(tokens, table):
    batch, hist = tokens.shape
    assert hist == HIST
    info = plsc.get_sparse_core_info()
    nc, ns = info.num_cores, info.num_subcores
    nw = nc * ns
    rows_per_w = batch // nw

    tok = tokens.astype(jnp.int32)

    split_blk = 512
    t0, t1 = pl.pallas_call(
        _split_body,
        grid=(batch // split_blk,),
        in_specs=[pl.BlockSpec((split_blk, HIST), lambda i: (i, 0))],
        out_specs=[
            pl.BlockSpec((split_blk, 128), lambda i: (i, 0)),
            pl.BlockSpec((split_blk, 128), lambda i: (i, 0)),
        ],
        out_shape=[
            jax.ShapeDtypeStruct((batch, 128), jnp.int32),
            jax.ShapeDtypeStruct((batch, 128), jnp.int32),
        ],
    )(tok)

    mesh = plsc.VectorSubcoreMesh(core_axis_name="c", subcore_axis_name="s")
    out128 = pl.kernel(
        functools.partial(_gather_body, rows_per_w=rows_per_w, nc=nc),
        mesh=mesh,
        out_type=jax.ShapeDtypeStruct((batch * HIST // 2, 128), jnp.float32),
        scratch_types=[
            pltpu.VMEM((rows_per_w, 128), jnp.int32),
            pltpu.VMEM((rows_per_w, 128), jnp.int32),
            pltpu.VMEM((2 * HIST, D), jnp.float32),
            pltpu.VMEM((HIST, 2 * D), jnp.float32),
            pltpu.SemaphoreType.DMA,
        ],
        compiler_params=pltpu.CompilerParams(use_tc_tiling_on_sc=False),
    )(t0, t1, table)

    return out128.reshape(batch, HIST, D)


# speedup vs baseline: 1.1136x; 1.1136x over previous
import functools

import jax
import jax.numpy as jnp
from jax import lax
from jax.experimental import pallas as pl
from jax.experimental.pallas import tpu as pltpu
from jax.experimental.pallas import tpu_sc as plsc

D = 64
SCALE = 8.0
HIST = 200
TAIL = HIST - 128


def _split_body(tok_ref, t0_ref, t1_ref):
    x = tok_ref[...]
    t0_ref[...] = x[:, :128]
    t1_ref[...] = x[:, HIST - 128:HIST]


def _gather_body(t0_hbm, t1_hbm, table_hbm, out_hbm, idx0_v, idx1_v, rows_v,
                 comp_v, sem, *, rows_per_w, nc):
    wid = lax.axis_index("s") * nc + lax.axis_index("c")
    b0 = pl.multiple_of(wid * rows_per_w, 8)
    pltpu.sync_copy(t0_hbm.at[pl.ds(b0, rows_per_w)], idx0_v)
    pltpu.sync_copy(t1_hbm.at[pl.ds(b0, rows_per_w)], idx1_v)

    def row_body(j, carry):
        cp0 = pltpu.async_copy(
            table_hbm.at[idx0_v.at[j]], rows_v.at[pl.ds(0, 128)], sem)
        cp1 = pltpu.async_copy(
            table_hbm.at[idx1_v.at[j, pl.ds(128 - TAIL, TAIL)]],
            rows_v.at[pl.ds(128, TAIL)], sem)
        cp0.wait()
        cp1.wait()

        def scale_body(r, carry2):
            for k in range(D // 16):
                sl = pl.ds(k * 16, 16)
                comp_v[r, sl] = rows_v[r, sl] * SCALE
            return carry2

        lax.fori_loop(0, HIST, scale_body, 0, unroll=4)

        off = pl.multiple_of((b0 + j) * HIST, 8)
        pltpu.sync_copy(comp_v, out_hbm.at[pl.ds(off, HIST)])
        return carry

    lax.fori_loop(0, rows_per_w, row_body, 0)


def kernel(tokens, table):
    batch, hist = tokens.shape
    assert hist == HIST
    info = plsc.get_sparse_core_info()
    nc, ns = info.num_cores, info.num_subcores
    nw = nc * ns
    rows_per_w = batch // nw

    tok = tokens.astype(jnp.int32)
    tpad = jnp.pad(table, ((0, 0), (0, 128 - D)))

    split_blk = 512
    t0, t1 = pl.pallas_call(
        _split_body,
        grid=(batch // split_blk,),
        in_specs=[pl.BlockSpec((split_blk, HIST), lambda i: (i, 0))],
        out_specs=[
            pl.BlockSpec((split_blk, 128), lambda i: (i, 0)),
            pl.BlockSpec((split_blk, 128), lambda i: (i, 0)),
        ],
        out_shape=[
            jax.ShapeDtypeStruct((batch, 128), jnp.int32),
            jax.ShapeDtypeStruct((batch, 128), jnp.int32),
        ],
    )(tok)

    mesh = plsc.VectorSubcoreMesh(core_axis_name="c", subcore_axis_name="s")
    flat = pl.kernel(
        functools.partial(_gather_body, rows_per_w=rows_per_w, nc=nc),
        mesh=mesh,
        out_type=jax.ShapeDtypeStruct((batch * HIST, D), jnp.float32),
        scratch_types=[
            pltpu.VMEM((rows_per_w, 128), jnp.int32),
            pltpu.VMEM((rows_per_w, 128), jnp.int32),
            pltpu.VMEM((HIST, 2 * D), jnp.float32),
            pltpu.VMEM((HIST, D), jnp.float32),
            pltpu.SemaphoreType.DMA,
        ],
        compiler_params=pltpu.CompilerParams(use_tc_tiling_on_sc=True),
    )(t0, t1, tpad)

    return flat.reshape(batch, HIST, D)


# speedup vs baseline: 1.4525x; 1.3044x over previous
import functools

import jax
import jax.numpy as jnp
from jax import lax
from jax.experimental import pallas as pl
from jax.experimental.pallas import tpu as pltpu
from jax.experimental.pallas import tpu_sc as plsc

D = 64
SCALE = 8.0
HIST = 200
TAIL = HIST - 128


def _split_body(tok_ref, t0_ref, t1_ref):
    x = tok_ref[...]
    t0_ref[...] = x[:, :128]
    t1_ref[...] = x[:, HIST - 128:HIST]


def _gather_body(t0_hbm, t1_hbm, table_hbm, out_hbm, idx0_v, idx1_v,
                 rows0_v, rows1_v, comp0_v, comp1_v, sg0, sg1, ss0, ss1,
                 *, rows_per_w, nc):
    wid = lax.axis_index("s") * nc + lax.axis_index("c")
    b0 = pl.multiple_of(wid * rows_per_w, 8)
    half = rows_per_w // 2

    rows = (rows0_v, rows1_v)
    comps = (comp0_v, comp1_v)
    sgs = (sg0, sg1)
    sss = (ss0, ss1)

    def fire_gather(j, p):
        pltpu.async_copy(
            table_hbm.at[idx0_v.at[j]], rows[p].at[pl.ds(0, 128)], sgs[p])
        pltpu.async_copy(
            table_hbm.at[idx1_v.at[j, pl.ds(128 - TAIL, TAIL)]],
            rows[p].at[pl.ds(128, TAIL)], sgs[p])

    def wait_gather(p):
        pltpu.make_async_copy(
            table_hbm.at[idx0_v.at[0]], rows[p].at[pl.ds(0, 128)],
            sgs[p]).wait()
        pltpu.make_async_copy(
            table_hbm.at[idx1_v.at[0, pl.ds(128 - TAIL, TAIL)]],
            rows[p].at[pl.ds(128, TAIL)], sgs[p]).wait()

    def run_half(h0, first):
        pltpu.sync_copy(t0_hbm.at[pl.ds(b0 + h0, half)], idx0_v)
        pltpu.sync_copy(t1_hbm.at[pl.ds(b0 + h0, half)], idx1_v)
        fire_gather(0, 0)
        fire_gather(1, 1)

        def pair_body(i2, carry):
            for p in range(2):
                j = 2 * i2 + p
                wait_gather(p)

                @pl.when(jnp.logical_or(i2 >= 1, jnp.logical_not(first)))
                def _():
                    pltpu.make_async_copy(
                        comps[p], out_hbm.at[pl.ds(0, HIST)], sss[p]).wait()

                def scale_body(r, carry2):
                    for k in range(D // 16):
                        sl = pl.ds(k * 16, 16)
                        comps[p][r, sl] = rows[p][r, sl] * SCALE
                    return carry2

                lax.fori_loop(0, HIST, scale_body, 0, unroll=4)

                off = pl.multiple_of((b0 + h0 + j) * HIST, 8)
                pltpu.async_copy(
                    comps[p], out_hbm.at[pl.ds(off, HIST)], sss[p])

                @pl.when(i2 < half // 2 - 1)
                def _():
                    fire_gather(j + 2, p)
            return carry

        lax.fori_loop(0, half // 2, pair_body, 0)

    run_half(0, True)
    run_half(half, False)

    for p in range(2):
        pltpu.make_async_copy(
            comps[p], out_hbm.at[pl.ds(0, HIST)], sss[p]).wait()


def kernel(tokens, table):
    batch, hist = tokens.shape
    assert hist == HIST
    info = plsc.get_sparse_core_info()
    nc, ns = info.num_cores, info.num_subcores
    nw = nc * ns
    rows_per_w = batch // nw

    tok = tokens.astype(jnp.int32)
    tpad = jnp.pad(table, ((0, 0), (0, 128 - D)))

    split_blk = 512
    t0, t1 = pl.pallas_call(
        _split_body,
        grid=(batch // split_blk,),
        in_specs=[pl.BlockSpec((split_blk, HIST), lambda i: (i, 0))],
        out_specs=[
            pl.BlockSpec((split_blk, 128), lambda i: (i, 0)),
            pl.BlockSpec((split_blk, 128), lambda i: (i, 0)),
        ],
        out_shape=[
            jax.ShapeDtypeStruct((batch, 128), jnp.int32),
            jax.ShapeDtypeStruct((batch, 128), jnp.int32),
        ],
    )(tok)

    mesh = plsc.VectorSubcoreMesh(core_axis_name="c", subcore_axis_name="s")
    flat = pl.kernel(
        functools.partial(_gather_body, rows_per_w=rows_per_w, nc=nc),
        mesh=mesh,
        out_type=jax.ShapeDtypeStruct((batch * HIST, D), jnp.float32),
        scratch_types=[
            pltpu.VMEM((rows_per_w // 2, 128), jnp.int32),
            pltpu.VMEM((rows_per_w // 2, 128), jnp.int32),
            pltpu.VMEM((HIST, 2 * D), jnp.float32),
            pltpu.VMEM((HIST, 2 * D), jnp.float32),
            pltpu.VMEM((HIST, D), jnp.float32),
            pltpu.VMEM((HIST, D), jnp.float32),
            pltpu.SemaphoreType.DMA,
            pltpu.SemaphoreType.DMA,
            pltpu.SemaphoreType.DMA,
            pltpu.SemaphoreType.DMA,
        ],
        compiler_params=pltpu.CompilerParams(use_tc_tiling_on_sc=True),
    )(t0, t1, tpad)

    return flat.reshape(batch, HIST, D)


# speedup vs baseline: 1.6535x; 1.1384x over previous
import functools

import jax
import jax.numpy as jnp
from jax import lax
from jax.experimental import pallas as pl
from jax.experimental.pallas import tpu as pltpu
from jax.experimental.pallas import tpu_sc as plsc

D = 64
SCALE = 8.0
HIST = 200
TAIL = HIST - 128


def _split_body(tok_ref, t0_ref, t1_ref):
    x = tok_ref[...]
    t0_ref[...] = x[:, :128]
    t1_ref[...] = x[:, HIST - 128:HIST]


def _gather_body(t0_hbm, t1_hbm, table_hbm, out_hbm, idx0_v, idx1_v,
                 rows0_v, rows1_v, comp0_v, comp1_v, sg0, sg1, st0, st1,
                 ss0, ss1, *, rows_per_w, nc):
    wid = lax.axis_index("s") * nc + lax.axis_index("c")
    b0 = pl.multiple_of(wid * rows_per_w, 8)
    half = rows_per_w // 2

    rows = (rows0_v, rows1_v)
    comps = (comp0_v, comp1_v)
    sgs = (sg0, sg1)
    sts = (st0, st1)
    sss = (ss0, ss1)

    def fire_gather(j, p):
        pltpu.async_copy(
            table_hbm.at[idx0_v.at[j]], rows[p].at[pl.ds(0, 128)], sgs[p])
        pltpu.async_copy(
            table_hbm.at[idx1_v.at[j, pl.ds(128 - TAIL, TAIL)]],
            rows[p].at[pl.ds(128, TAIL)], sts[p])

    def wait_gather_head(p):
        pltpu.make_async_copy(
            table_hbm.at[idx0_v.at[0]], rows[p].at[pl.ds(0, 128)],
            sgs[p]).wait()

    def wait_gather_tail(p):
        pltpu.make_async_copy(
            table_hbm.at[idx1_v.at[0, pl.ds(128 - TAIL, TAIL)]],
            rows[p].at[pl.ds(128, TAIL)], sts[p]).wait()

    def run_half(h0, first):
        pltpu.sync_copy(t0_hbm.at[pl.ds(b0 + h0, half)], idx0_v)
        pltpu.sync_copy(t1_hbm.at[pl.ds(b0 + h0, half)], idx1_v)
        fire_gather(0, 0)
        fire_gather(1, 1)

        def pair_body(i2, carry):
            for p in range(2):
                j = 2 * i2 + p
                wait_gather_head(p)

                @pl.when(jnp.logical_or(i2 >= 1, jnp.logical_not(first)))
                def _():
                    pltpu.make_async_copy(
                        comps[p], out_hbm.at[pl.ds(0, HIST)], sss[p]).wait()

                def scale_body(r, carry2):
                    for k in range(D // 16):
                        sl = pl.ds(k * 16, 16)
                        comps[p][r, sl] = rows[p][r, sl] * SCALE
                    return carry2

                lax.fori_loop(0, 128, scale_body, 0, unroll=8)
                wait_gather_tail(p)
                lax.fori_loop(128, HIST, scale_body, 0, unroll=8)

                off = pl.multiple_of((b0 + h0 + j) * HIST, 8)
                pltpu.async_copy(
                    comps[p], out_hbm.at[pl.ds(off, HIST)], sss[p])

                @pl.when(i2 < half // 2 - 1)
                def _():
                    fire_gather(j + 2, p)
            return carry

        lax.fori_loop(0, half // 2, pair_body, 0)

    run_half(0, True)
    run_half(half, False)

    for p in range(2):
        pltpu.make_async_copy(
            comps[p], out_hbm.at[pl.ds(0, HIST)], sss[p]).wait()


def kernel(tokens, table):
    batch, hist = tokens.shape
    assert hist == HIST
    info = plsc.get_sparse_core_info()
    nc, ns = info.num_cores, info.num_subcores
    nw = nc * ns
    rows_per_w = batch // nw

    tok = tokens.astype(jnp.int32)
    tpad = jnp.pad(table, ((0, 0), (0, 128 - D)))

    split_blk = 512
    t0, t1 = pl.pallas_call(
        _split_body,
        grid=(batch // split_blk,),
        in_specs=[pl.BlockSpec((split_blk, HIST), lambda i: (i, 0))],
        out_specs=[
            pl.BlockSpec((split_blk, 128), lambda i: (i, 0)),
            pl.BlockSpec((split_blk, 128), lambda i: (i, 0)),
        ],
        out_shape=[
            jax.ShapeDtypeStruct((batch, 128), jnp.int32),
            jax.ShapeDtypeStruct((batch, 128), jnp.int32),
        ],
    )(tok)

    mesh = plsc.VectorSubcoreMesh(core_axis_name="c", subcore_axis_name="s")
    flat = pl.kernel(
        functools.partial(_gather_body, rows_per_w=rows_per_w, nc=nc),
        mesh=mesh,
        out_type=jax.ShapeDtypeStruct((batch * HIST, D), jnp.float32),
        scratch_types=[
            pltpu.VMEM((rows_per_w // 2, 128), jnp.int32),
            pltpu.VMEM((rows_per_w // 2, 128), jnp.int32),
            pltpu.VMEM((HIST, 2 * D), jnp.float32),
            pltpu.VMEM((HIST, 2 * D), jnp.float32),
            pltpu.VMEM((HIST, D), jnp.float32),
            pltpu.VMEM((HIST, D), jnp.float32),
            pltpu.SemaphoreType.DMA,
            pltpu.SemaphoreType.DMA,
            pltpu.SemaphoreType.DMA,
            pltpu.SemaphoreType.DMA,
            pltpu.SemaphoreType.DMA,
            pltpu.SemaphoreType.DMA,
        ],
        compiler_params=pltpu.CompilerParams(use_tc_tiling_on_sc=True),
    )(t0, t1, tpad)

    return flat.reshape(batch, HIST, D)
